# Initial kernel scaffold; baseline (speedup 1.0000x reference)
import jax
import jax.numpy as jnp
from jax.experimental import pallas as pl


def kernel(x_individual, x_facility, ei_interacts, ei_visits, ei_visited_by, ei_connects, params):
    raise NotImplementedError("write your pallas kernel here")


# speedup vs baseline: 2.4899x; 2.4899x over previous
import functools

import jax
import jax.numpy as jnp
from jax import lax
from jax.experimental import pallas as pl
from jax.experimental.pallas import tpu as pltpu
from jax.experimental.pallas import tpu_sc as plsc

H = 128
L = 16
NSUB = 16
NCORE = 2
B = 512
G = 128
CB = 656

IND_N, FAC_N = 50000, 5000
IND_NP, FAC_NP = 51200, 5120
IND_CHUNK, FAC_CHUNK = 12800, 2560
ACC_ROWS = 12816
GARBAGE = 12800

E_II, E_FI, E_IF, E_FF = 507904, 212992, 212992, 65536

f32 = jnp.float32
i32 = jnp.int32


def _zeros16():
    return jnp.zeros((L,), f32)


def _emit_counts(c, s, dst_e, out, npad, nblocks, dstv, cnt_local, tmpv,
                 accbuf, slab):
    w = c * NSUB + s
    lax.fori_loop(
        0, npad // L,
        lambda i, _: (cnt_local.__setitem__(pl.ds(i * L, L), _zeros16()), 0)[1],
        0)
    tile_e = dst_e.shape[0] // (NCORE * NSUB)
    tbase = w * tile_e
    ones = jnp.ones((L,), f32)

    def blk(bi, _):
        pltpu.sync_copy(dst_e.at[pl.ds(tbase + bi * B, B)], dstv)
        for k in range(B // L):
            dv = dstv[pl.ds(k * L, L)]
            plsc.addupdate_scatter(cnt_local, [dv], ones, mask=dv >= 0)
        return 0

    lax.fori_loop(0, nblocks, blk, 0)
    pltpu.sync_copy(cnt_local.at[pl.ds(0, npad)],
                    slab.at[pl.ds(s * IND_NP, npad)])
    plsc.subcore_barrier()
    sl = npad // NSUB
    lax.fori_loop(
        0, sl // L,
        lambda i, _: (accbuf.__setitem__(pl.ds(i * L, L), _zeros16()), 0)[1],
        0)
    for t in range(NSUB):
        pltpu.sync_copy(slab.at[pl.ds(t * IND_NP + s * sl, sl)],
                        tmpv.at[pl.ds(0, sl)])

        def addb(i, _):
            accbuf[pl.ds(i * L, L)] = (accbuf[pl.ds(i * L, L)]
                                       + tmpv[pl.ds(i * L, L)])
            return 0

        lax.fori_loop(0, sl // L, addb, 0)
    pltpu.sync_copy(accbuf.at[pl.ds(0, sl)],
                    out.at[pl.ds(c * npad + s * sl, sl)])
    plsc.subcore_barrier()


def _counts_call(d_ii, d_fi, d_if, d_ff):
    mesh = plsc.VectorSubcoreMesh(core_axis_name="c", subcore_axis_name="s")

    @functools.partial(
        pl.kernel,
        out_type=(jax.ShapeDtypeStruct((NCORE * IND_NP,), f32),
                  jax.ShapeDtypeStruct((NCORE * IND_NP,), f32),
                  jax.ShapeDtypeStruct((NCORE * FAC_NP,), f32),
                  jax.ShapeDtypeStruct((NCORE * FAC_NP,), f32)),
        mesh=mesh,
        scratch_types=[
            pltpu.VMEM((B,), i32),
            pltpu.VMEM((IND_NP,), f32),
            pltpu.VMEM((IND_NP // NSUB,), f32),
            pltpu.VMEM((IND_NP // NSUB,), f32),
            pltpu.VMEM_SHARED((NSUB * IND_NP,), f32),
        ],
        compiler_params=pltpu.CompilerParams(needs_layout_passes=False),
    )
    def k(d_ii_h, d_fi_h, d_if_h, d_ff_h, o_ii, o_fi, o_if, o_ff,
          dstv, cnt_local, tmpv, accbuf, slab):
        c = lax.axis_index("c")
        s = lax.axis_index("s")
        _emit_counts(c, s, d_ii_h, o_ii, IND_NP, E_II // 32 // B,
                     dstv, cnt_local, tmpv, accbuf, slab)
        _emit_counts(c, s, d_fi_h, o_fi, IND_NP, E_FI // 32 // B,
                     dstv, cnt_local, tmpv, accbuf, slab)
        _emit_counts(c, s, d_if_h, o_if, FAC_NP, E_IF // 32 // B,
                     dstv, cnt_local, tmpv, accbuf, slab)
        _emit_counts(c, s, d_ff_h, o_ff, FAC_NP, E_FF // 32 // B,
                     dstv, cnt_local, tmpv, accbuf, slab)

    return k(d_ii, d_fi, d_if, d_ff)


def _zero_fill_rows(rows):
    def zb(g, _):
        for k in range(H // L):
            rows[g, pl.ds(k * L, L)] = _zeros16()
        return 0
    lax.fori_loop(0, G, zb, 0)


def _zero_rows(zeros_v, acc, base, count):
    off = 0
    while off < count:
        blk = min(G, count - off)
        pltpu.sync_copy(zeros_v.at[pl.ds(0, blk)],
                        acc.at[pl.ds(base + off, blk)])
        off += blk


def _emit_job(c, s, hsrc, src_e, dst_e, out, nchunks, chunk, nblocks,
              srcv, dstv, cbs, cbd, gidx, gdst, rows, acc, sem):
    tile_e = src_e.shape[0] // NSUB
    span = chunk // NSUB
    lane = lax.iota(i32, L)

    def chunk_body(r, _):
        lo = (nchunks * c + r) * chunk
        hi = lo + chunk
        _zero_fill_rows(rows)
        _zero_rows(rows, acc, s * span, span)
        plsc.subcore_barrier()
        tbase = s * tile_e

        def drain(n):
            for k in range(G // L):
                gidx[pl.ds(k * L, L)] = cbs[pl.ds(k * L, L)]
                gdst[pl.ds(k * L, L)] = cbd[pl.ds(k * L, L)]
            pltpu.async_copy(hsrc.at[gidx], rows, sem).wait()
            pltpu.sync_copy(rows, acc.at[gdst], add=True)
            for k in range((CB - G) // L):
                cbs[pl.ds(k * L, L)] = cbs[pl.ds(G + k * L, L)]
                cbd[pl.ds(k * L, L)] = cbd[pl.ds(G + k * L, L)]
            return n - G

        def block_body(bi, n):
            ebase = tbase + bi * B
            pltpu.sync_copy(src_e.at[pl.ds(ebase, B)], srcv)
            pltpu.sync_copy(dst_e.at[pl.ds(ebase, B)], dstv)
            for k in range(B // L):
                sv = srcv[pl.ds(k * L, L)]
                dv = dstv[pl.ds(k * L, L)]
                m = (dv >= lo) & (dv < hi)
                plsc.store_compressed(cbs.at[pl.ds(n, L)], sv, mask=m)
                plsc.store_compressed(cbd.at[pl.ds(n, L)], dv - lo, mask=m)
                n = n + jnp.sum(m.astype(i32))
            return lax.while_loop(lambda q: q >= G, drain, n)

        n = lax.fori_loop(0, nblocks, block_body, jnp.asarray(0, i32))
        for k in range(G // L):
            m = (k * L + lane) < n
            gidx[pl.ds(k * L, L)] = jnp.where(m, cbs[pl.ds(k * L, L)],
                                              jnp.zeros((L,), i32))
            gdst[pl.ds(k * L, L)] = jnp.where(m, cbd[pl.ds(k * L, L)],
                                              jnp.full((L,), GARBAGE, i32))
        pltpu.async_copy(hsrc.at[gidx], rows, sem).wait()
        pltpu.sync_copy(rows, acc.at[gdst], add=True)
        plsc.subcore_barrier()
        pltpu.sync_copy(acc.at[pl.ds(s * span, span)],
                        out.at[pl.ds(lo + s * span, span)])
        plsc.subcore_barrier()
        return 0

    lax.fori_loop(0, nchunks, chunk_body, 0)


def _agg_call(h_i, h_f, s_ii, d_ii, s_fi, d_fi, s_if, d_if, s_ff, d_ff):
    mesh = plsc.VectorSubcoreMesh(core_axis_name="c", subcore_axis_name="s")

    @functools.partial(
        pl.kernel,
        out_type=(jax.ShapeDtypeStruct((IND_NP, H), f32),
                  jax.ShapeDtypeStruct((IND_NP, H), f32),
                  jax.ShapeDtypeStruct((FAC_NP, H), f32),
                  jax.ShapeDtypeStruct((FAC_NP, H), f32)),
        mesh=mesh,
        scratch_types=[
            pltpu.VMEM((B,), i32),
            pltpu.VMEM((B,), i32),
            pltpu.VMEM((CB,), i32),
            pltpu.VMEM((CB,), i32),
            pltpu.VMEM((G,), i32),
            pltpu.VMEM((G,), i32),
            pltpu.VMEM((G, H), f32),
            pltpu.VMEM_SHARED((ACC_ROWS, H), f32),
            pltpu.SemaphoreType.DMA,
        ],
        compiler_params=pltpu.CompilerParams(needs_layout_passes=False),
    )
    def k(hi_h, hf_h, sii, dii, sfi, dfi, sif, dif, sff, dff,
          a_ii, a_fi, a_if, a_ff,
          srcv, dstv, cbs, cbd, gidx, gdst, rows, acc, sem):
        c = lax.axis_index("c")
        s = lax.axis_index("s")
        scr = (srcv, dstv, cbs, cbd, gidx, gdst, rows, acc, sem)
        _emit_job(c, s, hi_h, sii, dii, a_ii, 2, IND_CHUNK, E_II // NSUB // B,
                  *scr)
        _emit_job(c, s, hf_h, sfi, dfi, a_fi, 2, IND_CHUNK, E_FI // NSUB // B,
                  *scr)
        _emit_job(c, s, hi_h, sif, dif, a_if, 1, FAC_CHUNK, E_IF // NSUB // B,
                  *scr)
        _emit_job(c, s, hf_h, sff, dff, a_ff, 1, FAC_CHUNK, E_FF // NSUB // B,
                  *scr)

    return k(h_i, h_f, s_ii, d_ii, s_fi, d_fi, s_if, d_if, s_ff, d_ff)


def _proj_body(x_ref, w_ref, b_ref, out_ref):
    out_ref[...] = (jnp.dot(x_ref[...], w_ref[...],
                            preferred_element_type=f32) + b_ref[...])


def _proj_call(x, w, b):
    n = x.shape[0]
    r = 512
    return pl.pallas_call(
        _proj_body,
        grid=(n // r,),
        in_specs=[pl.BlockSpec((r, 8), lambda i: (i, 0)),
                  pl.BlockSpec((8, H), lambda i: (0, 0)),
                  pl.BlockSpec((1, H), lambda i: (0, 0))],
        out_specs=pl.BlockSpec((r, H), lambda i: (i, 0)),
        out_shape=jax.ShapeDtypeStruct((n, H), f32),
    )(x, w, b)


def _dense_body(h_ref, aa_ref, ab_ref, ia_ref, ib_ref, wla_ref, wlb_ref,
                wr_ref, bias_ref, g_ref, bln_ref, out_ref):
    h = h_ref[...]
    t = (jnp.dot(aa_ref[...] * ia_ref[...], wla_ref[...],
                 preferred_element_type=f32)
         + jnp.dot(ab_ref[...] * ib_ref[...], wlb_ref[...],
                   preferred_element_type=f32)
         + jnp.dot(h, wr_ref[...], preferred_element_type=f32)
         + bias_ref[...])
    mu = jnp.mean(t, axis=-1, keepdims=True)
    d = t - mu
    var = jnp.mean(d * d, axis=-1, keepdims=True)
    ln = g_ref[...] * d * lax.rsqrt(var + 1e-5) + bln_ref[...]
    out_ref[...] = jnp.maximum(h + ln, 0.0)


def _dense_call(h, aa, ab, ia, ib, wla, wlb, wr, bias, g, bln):
    n = h.shape[0]
    r = 512
    row = lambda i: (i, 0)
    fix = lambda i: (0, 0)
    return pl.pallas_call(
        _dense_body,
        grid=(n // r,),
        in_specs=[pl.BlockSpec((r, H), row), pl.BlockSpec((r, H), row),
                  pl.BlockSpec((r, H), row), pl.BlockSpec((r, 1), row),
                  pl.BlockSpec((r, 1), row), pl.BlockSpec((H, H), fix),
                  pl.BlockSpec((H, H), fix), pl.BlockSpec((H, H), fix),
                  pl.BlockSpec((1, H), fix), pl.BlockSpec((1, H), fix),
                  pl.BlockSpec((1, H), fix)],
        out_specs=pl.BlockSpec((r, H), row),
        out_shape=jax.ShapeDtypeStruct((n, H), f32),
    )(h, aa, ab, ia, ib, wla, wlb, wr, bias, g, bln)


def _pad_edges(src, dst, epad):
    e = src.shape[0]
    src = jnp.concatenate([src, jnp.zeros((epad - e,), i32)])
    dst = jnp.concatenate([dst, jnp.full((epad - e,), -1, i32)])
    return src, dst


def kernel(x_individual, x_facility, ei_interacts, ei_visits, ei_visited_by,
           ei_connects, params):
    p = params
    xi = jnp.zeros((IND_NP, 8), f32).at[:IND_N, :5].set(x_individual)
    xf = jnp.zeros((FAC_NP, 8), f32).at[:FAC_N, :3].set(x_facility)
    wi = jnp.zeros((8, H), f32).at[:5].set(p['ind_proj_W'])
    wf = jnp.zeros((8, H), f32).at[:3].set(p['fac_proj_W'])

    s_ii, d_ii = _pad_edges(ei_interacts[0], ei_interacts[1], E_II)
    s_fi, d_fi = _pad_edges(ei_visited_by[0], ei_visited_by[1], E_FI)
    s_if, d_if = _pad_edges(ei_visits[0], ei_visits[1], E_IF)
    s_ff, d_ff = _pad_edges(ei_connects[0], ei_connects[1], E_FF)

    c_ii, c_fi, c_if, c_ff = _counts_call(d_ii, d_fi, d_if, d_ff)
    inv = lambda c, npad: (1.0 / jnp.maximum(
        c.reshape(NCORE, npad).sum(0), 1.0))[:, None]
    inv_ii, inv_fi = inv(c_ii, IND_NP), inv(c_fi, IND_NP)
    inv_if, inv_ff = inv(c_if, FAC_NP), inv(c_ff, FAC_NP)

    h_i = _proj_call(xi, wi, p['ind_proj_b'][None])
    h_f = _proj_call(xf, wf, p['fac_proj_b'][None])

    for l in range(3):
        a_ii, a_fi, a_if, a_ff = _agg_call(
            h_i, h_f, s_ii, d_ii, s_fi, d_fi, s_if, d_if, s_ff, d_ff)
        h_i = _dense_call(
            h_i, a_ii, a_fi, inv_ii, inv_fi,
            p[f'ii{l}_Wl'], p[f'fi{l}_Wl'], p[f'ii{l}_Wr'] + p[f'fi{l}_Wr'],
            (p[f'ii{l}_bl'] + p[f'fi{l}_bl'])[None],
            p[f'ln_ind{l}_g'][None], p[f'ln_ind{l}_b'][None])
        h_f = _dense_call(
            h_f, a_if, a_ff, inv_if, inv_ff,
            p[f'if{l}_Wl'], p[f'ff{l}_Wl'], p[f'if{l}_Wr'] + p[f'ff{l}_Wr'],
            (p[f'if{l}_bl'] + p[f'ff{l}_bl'])[None],
            p[f'ln_fac{l}_g'][None], p[f'ln_fac{l}_b'][None])

    return h_i[:IND_N], h_f[:FAC_N]


# speedup vs baseline: 2.8690x; 1.1522x over previous
import functools

import jax
import jax.numpy as jnp
from jax import lax
from jax.experimental import pallas as pl
from jax.experimental.pallas import tpu as pltpu
from jax.experimental.pallas import tpu_sc as plsc

H = 128
L = 16
NSUB = 16
NCORE = 2
B = 512
G = 128
CB = 656

IND_N, FAC_N = 50000, 5000
IND_NP, FAC_NP = 51200, 5120
IND_CHUNK, FAC_CHUNK = 12800, 2560
ACC_ROWS = 12816
GARBAGE = 12800

E_II, E_FI, E_IF, E_FF = 507904, 212992, 212992, 65536

f32 = jnp.float32
i32 = jnp.int32


def _zeros16():
    return jnp.zeros((L,), f32)


def _emit_counts(c, s, dst_e, out, npad, nblocks, dstv, cnt_local, tmpv,
                 accbuf, slab):
    w = c * NSUB + s
    lax.fori_loop(
        0, npad // L,
        lambda i, _: (cnt_local.__setitem__(pl.ds(i * L, L), _zeros16()), 0)[1],
        0)
    tile_e = dst_e.shape[0] // (NCORE * NSUB)
    tbase = w * tile_e
    ones = jnp.ones((L,), f32)

    def blk(bi, _):
        pltpu.sync_copy(dst_e.at[pl.ds(tbase + bi * B, B)], dstv)
        for k in range(B // L):
            dv = dstv[pl.ds(k * L, L)]
            plsc.addupdate_scatter(cnt_local, [dv], ones, mask=dv >= 0)
        return 0

    lax.fori_loop(0, nblocks, blk, 0)
    pltpu.sync_copy(cnt_local.at[pl.ds(0, npad)],
                    slab.at[pl.ds(s * IND_NP, npad)])
    plsc.subcore_barrier()
    sl = npad // NSUB
    lax.fori_loop(
        0, sl // L,
        lambda i, _: (accbuf.__setitem__(pl.ds(i * L, L), _zeros16()), 0)[1],
        0)
    for t in range(NSUB):
        pltpu.sync_copy(slab.at[pl.ds(t * IND_NP + s * sl, sl)],
                        tmpv.at[pl.ds(0, sl)])

        def addb(i, _):
            accbuf[pl.ds(i * L, L)] = (accbuf[pl.ds(i * L, L)]
                                       + tmpv[pl.ds(i * L, L)])
            return 0

        lax.fori_loop(0, sl // L, addb, 0)
    pltpu.sync_copy(accbuf.at[pl.ds(0, sl)],
                    out.at[pl.ds(c * npad + s * sl, sl)])
    plsc.subcore_barrier()


def _counts_call(d_ii, d_fi, d_if, d_ff):
    mesh = plsc.VectorSubcoreMesh(core_axis_name="c", subcore_axis_name="s")

    @functools.partial(
        pl.kernel,
        out_type=(jax.ShapeDtypeStruct((NCORE * IND_NP,), f32),
                  jax.ShapeDtypeStruct((NCORE * IND_NP,), f32),
                  jax.ShapeDtypeStruct((NCORE * FAC_NP,), f32),
                  jax.ShapeDtypeStruct((NCORE * FAC_NP,), f32)),
        mesh=mesh,
        scratch_types=[
            pltpu.VMEM((B,), i32),
            pltpu.VMEM((IND_NP,), f32),
            pltpu.VMEM((IND_NP // NSUB,), f32),
            pltpu.VMEM((IND_NP // NSUB,), f32),
            pltpu.VMEM_SHARED((NSUB * IND_NP,), f32),
        ],
        compiler_params=pltpu.CompilerParams(needs_layout_passes=False),
    )
    def k(d_ii_h, d_fi_h, d_if_h, d_ff_h, o_ii, o_fi, o_if, o_ff,
          dstv, cnt_local, tmpv, accbuf, slab):
        c = lax.axis_index("c")
        s = lax.axis_index("s")
        _emit_counts(c, s, d_ii_h, o_ii, IND_NP, E_II // 32 // B,
                     dstv, cnt_local, tmpv, accbuf, slab)
        _emit_counts(c, s, d_fi_h, o_fi, IND_NP, E_FI // 32 // B,
                     dstv, cnt_local, tmpv, accbuf, slab)
        _emit_counts(c, s, d_if_h, o_if, FAC_NP, E_IF // 32 // B,
                     dstv, cnt_local, tmpv, accbuf, slab)
        _emit_counts(c, s, d_ff_h, o_ff, FAC_NP, E_FF // 32 // B,
                     dstv, cnt_local, tmpv, accbuf, slab)

    return k(d_ii, d_fi, d_if, d_ff)


def _zero_fill_rows(rows):
    def zb(g, _):
        for k in range(H // L):
            rows[g, pl.ds(k * L, L)] = _zeros16()
        return 0
    lax.fori_loop(0, G, zb, 0)


def _zero_rows(zeros_v, acc, base, count):
    off = 0
    while off < count:
        blk = min(G, count - off)
        pltpu.sync_copy(zeros_v.at[pl.ds(0, blk)],
                        acc.at[pl.ds(base + off, blk)])
        off += blk


def _emit_job(c, s, hsrc, src_e, dst_e, out, nchunks, chunk, nblocks,
              srcv, dstv, srcv2, dstv2, cbs, cbd, gidx, gdst, rows, acc,
              sem, semla, semlb):
    tile_e = src_e.shape[0] // NSUB
    span = chunk // NSUB
    lane = lax.iota(i32, L)
    npairs = nblocks // 2

    def chunk_body(r, _):
        lo = (nchunks * c + r) * chunk
        hi = lo + chunk
        _zero_fill_rows(rows)
        _zero_rows(rows, acc, s * span, span)
        plsc.subcore_barrier()
        tbase = s * tile_e

        def drain(n):
            for k in range(G // L):
                gidx[pl.ds(k * L, L)] = cbs[pl.ds(k * L, L)]
                gdst[pl.ds(k * L, L)] = cbd[pl.ds(k * L, L)]
            pltpu.async_copy(hsrc.at[gidx], rows, sem).wait()
            pltpu.sync_copy(rows, acc.at[gdst], add=True)
            for k in range((CB - G) // L):
                cbs[pl.ds(k * L, L)] = cbs[pl.ds(G + k * L, L)]
                cbd[pl.ds(k * L, L)] = cbd[pl.ds(G + k * L, L)]
            return n - G

        def compact(sbuf, dbuf, n):
            for k in range(B // L):
                sv = sbuf[pl.ds(k * L, L)]
                dv = dbuf[pl.ds(k * L, L)]
                m = (dv >= lo) & (dv < hi)
                plsc.store_compressed(cbs.at[pl.ds(n, L)], sv, mask=m)
                plsc.store_compressed(cbd.at[pl.ds(n, L)], dv - lo, mask=m)
                n = n + jnp.sum(m.astype(i32))
            return lax.while_loop(lambda q: q >= G, drain, n)

        def issue_load(ebase, sbuf, dbuf, seml):
            pltpu.async_copy(src_e.at[pl.ds(ebase, B)], sbuf, seml)
            pltpu.async_copy(dst_e.at[pl.ds(ebase, B)], dbuf, seml)

        def wait_load(sbuf, dbuf, seml):
            pltpu.make_async_copy(src_e.at[pl.ds(0, B)], sbuf, seml).wait()
            pltpu.make_async_copy(dst_e.at[pl.ds(0, B)], dbuf, seml).wait()

        issue_load(tbase, srcv, dstv, semla)

        def pair_body(bi, n):
            issue_load(tbase + (2 * bi + 1) * B, srcv2, dstv2, semlb)
            wait_load(srcv, dstv, semla)
            n = compact(srcv, dstv, n)

            @pl.when(bi + 1 < npairs)
            def _():
                issue_load(tbase + (2 * bi + 2) * B, srcv, dstv, semla)

            wait_load(srcv2, dstv2, semlb)
            return compact(srcv2, dstv2, n)

        n = lax.fori_loop(0, npairs, pair_body, jnp.asarray(0, i32))
        for k in range(G // L):
            m = (k * L + lane) < n
            gidx[pl.ds(k * L, L)] = jnp.where(m, cbs[pl.ds(k * L, L)],
                                              jnp.zeros((L,), i32))
            gdst[pl.ds(k * L, L)] = jnp.where(m, cbd[pl.ds(k * L, L)],
                                              jnp.full((L,), GARBAGE, i32))
        pltpu.async_copy(hsrc.at[gidx], rows, sem).wait()
        pltpu.sync_copy(rows, acc.at[gdst], add=True)
        plsc.subcore_barrier()
        pltpu.sync_copy(acc.at[pl.ds(s * span, span)],
                        out.at[pl.ds(lo + s * span, span)])
        plsc.subcore_barrier()
        return 0

    lax.fori_loop(0, nchunks, chunk_body, 0)


def _agg_call(h_i, h_f, s_ii, d_ii, s_fi, d_fi, s_if, d_if, s_ff, d_ff):
    mesh = plsc.VectorSubcoreMesh(core_axis_name="c", subcore_axis_name="s")

    @functools.partial(
        pl.kernel,
        out_type=(jax.ShapeDtypeStruct((IND_NP, H), f32),
                  jax.ShapeDtypeStruct((IND_NP, H), f32),
                  jax.ShapeDtypeStruct((FAC_NP, H), f32),
                  jax.ShapeDtypeStruct((FAC_NP, H), f32)),
        mesh=mesh,
        scratch_types=[
            pltpu.VMEM((B,), i32),
            pltpu.VMEM((B,), i32),
            pltpu.VMEM((B,), i32),
            pltpu.VMEM((B,), i32),
            pltpu.VMEM((CB,), i32),
            pltpu.VMEM((CB,), i32),
            pltpu.VMEM((G,), i32),
            pltpu.VMEM((G,), i32),
            pltpu.VMEM((G, H), f32),
            pltpu.VMEM_SHARED((ACC_ROWS, H), f32),
            pltpu.SemaphoreType.DMA,
            pltpu.SemaphoreType.DMA,
            pltpu.SemaphoreType.DMA,
        ],
        compiler_params=pltpu.CompilerParams(needs_layout_passes=False),
    )
    def k(hi_h, hf_h, sii, dii, sfi, dfi, sif, dif, sff, dff,
          a_ii, a_fi, a_if, a_ff,
          srcv, dstv, srcv2, dstv2, cbs, cbd, gidx, gdst, rows, acc,
          sem, semla, semlb):
        c = lax.axis_index("c")
        s = lax.axis_index("s")
        scr = (srcv, dstv, srcv2, dstv2, cbs, cbd, gidx, gdst, rows, acc,
               sem, semla, semlb)
        _emit_job(c, s, hi_h, sii, dii, a_ii, 2, IND_CHUNK, E_II // NSUB // B,
                  *scr)
        _emit_job(c, s, hf_h, sfi, dfi, a_fi, 2, IND_CHUNK, E_FI // NSUB // B,
                  *scr)
        _emit_job(c, s, hi_h, sif, dif, a_if, 1, FAC_CHUNK, E_IF // NSUB // B,
                  *scr)
        _emit_job(c, s, hf_h, sff, dff, a_ff, 1, FAC_CHUNK, E_FF // NSUB // B,
                  *scr)

    return k(h_i, h_f, s_ii, d_ii, s_fi, d_fi, s_if, d_if, s_ff, d_ff)


def _proj_body(x_ref, w_ref, b_ref, out_ref):
    out_ref[...] = (jnp.dot(x_ref[...], w_ref[...],
                            preferred_element_type=f32) + b_ref[...])


def _proj_call(x, w, b):
    n = x.shape[0]
    r = 512
    return pl.pallas_call(
        _proj_body,
        grid=(n // r,),
        in_specs=[pl.BlockSpec((r, 8), lambda i: (i, 0)),
                  pl.BlockSpec((8, H), lambda i: (0, 0)),
                  pl.BlockSpec((1, H), lambda i: (0, 0))],
        out_specs=pl.BlockSpec((r, H), lambda i: (i, 0)),
        out_shape=jax.ShapeDtypeStruct((n, H), f32),
    )(x, w, b)


def _dense_body(h_ref, aa_ref, ab_ref, ia_ref, ib_ref, wla_ref, wlb_ref,
                wr_ref, bias_ref, g_ref, bln_ref, out_ref):
    h = h_ref[...]
    t = (jnp.dot(aa_ref[...] * ia_ref[...], wla_ref[...],
                 preferred_element_type=f32)
         + jnp.dot(ab_ref[...] * ib_ref[...], wlb_ref[...],
                   preferred_element_type=f32)
         + jnp.dot(h, wr_ref[...], preferred_element_type=f32)
         + bias_ref[...])
    mu = jnp.mean(t, axis=-1, keepdims=True)
    d = t - mu
    var = jnp.mean(d * d, axis=-1, keepdims=True)
    ln = g_ref[...] * d * lax.rsqrt(var + 1e-5) + bln_ref[...]
    out_ref[...] = jnp.maximum(h + ln, 0.0)


def _dense_call(h, aa, ab, ia, ib, wla, wlb, wr, bias, g, bln):
    n = h.shape[0]
    r = 512
    row = lambda i: (i, 0)
    fix = lambda i: (0, 0)
    return pl.pallas_call(
        _dense_body,
        grid=(n // r,),
        in_specs=[pl.BlockSpec((r, H), row), pl.BlockSpec((r, H), row),
                  pl.BlockSpec((r, H), row), pl.BlockSpec((r, 1), row),
                  pl.BlockSpec((r, 1), row), pl.BlockSpec((H, H), fix),
                  pl.BlockSpec((H, H), fix), pl.BlockSpec((H, H), fix),
                  pl.BlockSpec((1, H), fix), pl.BlockSpec((1, H), fix),
                  pl.BlockSpec((1, H), fix)],
        out_specs=pl.BlockSpec((r, H), row),
        out_shape=jax.ShapeDtypeStruct((n, H), f32),
    )(h, aa, ab, ia, ib, wla, wlb, wr, bias, g, bln)


def _pad_edges(src, dst, epad):
    e = src.shape[0]
    src = jnp.concatenate([src, jnp.zeros((epad - e,), i32)])
    dst = jnp.concatenate([dst, jnp.full((epad - e,), -1, i32)])
    return src, dst


def kernel(x_individual, x_facility, ei_interacts, ei_visits, ei_visited_by,
           ei_connects, params):
    p = params
    xi = jnp.zeros((IND_NP, 8), f32).at[:IND_N, :5].set(x_individual)
    xf = jnp.zeros((FAC_NP, 8), f32).at[:FAC_N, :3].set(x_facility)
    wi = jnp.zeros((8, H), f32).at[:5].set(p['ind_proj_W'])
    wf = jnp.zeros((8, H), f32).at[:3].set(p['fac_proj_W'])

    s_ii, d_ii = _pad_edges(ei_interacts[0], ei_interacts[1], E_II)
    s_fi, d_fi = _pad_edges(ei_visited_by[0], ei_visited_by[1], E_FI)
    s_if, d_if = _pad_edges(ei_visits[0], ei_visits[1], E_IF)
    s_ff, d_ff = _pad_edges(ei_connects[0], ei_connects[1], E_FF)

    c_ii, c_fi, c_if, c_ff = _counts_call(d_ii, d_fi, d_if, d_ff)
    inv = lambda c, npad: (1.0 / jnp.maximum(
        c.reshape(NCORE, npad).sum(0), 1.0))[:, None]
    inv_ii, inv_fi = inv(c_ii, IND_NP), inv(c_fi, IND_NP)
    inv_if, inv_ff = inv(c_if, FAC_NP), inv(c_ff, FAC_NP)

    h_i = _proj_call(xi, wi, p['ind_proj_b'][None])
    h_f = _proj_call(xf, wf, p['fac_proj_b'][None])

    for l in range(3):
        a_ii, a_fi, a_if, a_ff = _agg_call(
            h_i, h_f, s_ii, d_ii, s_fi, d_fi, s_if, d_if, s_ff, d_ff)
        h_i = _dense_call(
            h_i, a_ii, a_fi, inv_ii, inv_fi,
            p[f'ii{l}_Wl'], p[f'fi{l}_Wl'], p[f'ii{l}_Wr'] + p[f'fi{l}_Wr'],
            (p[f'ii{l}_bl'] + p[f'fi{l}_bl'])[None],
            p[f'ln_ind{l}_g'][None], p[f'ln_ind{l}_b'][None])
        h_f = _dense_call(
            h_f, a_if, a_ff, inv_if, inv_ff,
            p[f'if{l}_Wl'], p[f'ff{l}_Wl'], p[f'if{l}_Wr'] + p[f'ff{l}_Wr'],
            (p[f'if{l}_bl'] + p[f'ff{l}_bl'])[None],
            p[f'ln_fac{l}_g'][None], p[f'ln_fac{l}_b'][None])

    return h_i[:IND_N], h_f[:FAC_N]


# speedup vs baseline: 3.0496x; 1.0630x over previous
import functools

import jax
import jax.numpy as jnp
from jax import lax
from jax.experimental import pallas as pl
from jax.experimental.pallas import tpu as pltpu
from jax.experimental.pallas import tpu_sc as plsc

H = 128
L = 16
NSUB = 16
NCORE = 2
B = 512
G = 128
CB = 656

IND_N, FAC_N = 50000, 5000
IND_NP, FAC_NP = 51200, 5120
IND_CHUNK, FAC_CHUNK = 12800, 2560
ACC_ROWS = 12816
GARBAGE = 12800

E_II, E_FI, E_IF, E_FF = 507904, 212992, 212992, 65536

f32 = jnp.float32
i32 = jnp.int32


def _zeros16():
    return jnp.zeros((L,), f32)


def _emit_counts(c, s, dst_e, out, npad, nblocks, dstv, cnt_local, tmpv,
                 accbuf, slab):
    w = c * NSUB + s
    lax.fori_loop(
        0, npad // L,
        lambda i, _: (cnt_local.__setitem__(pl.ds(i * L, L), _zeros16()), 0)[1],
        0)
    tile_e = dst_e.shape[0] // (NCORE * NSUB)
    tbase = w * tile_e
    ones = jnp.ones((L,), f32)

    def blk(bi, _):
        pltpu.sync_copy(dst_e.at[pl.ds(tbase + bi * B, B)], dstv)
        for k in range(B // L):
            dv = dstv[pl.ds(k * L, L)]
            plsc.addupdate_scatter(cnt_local, [dv], ones, mask=dv >= 0)
        return 0

    lax.fori_loop(0, nblocks, blk, 0)
    pltpu.sync_copy(cnt_local.at[pl.ds(0, npad)],
                    slab.at[pl.ds(s * IND_NP, npad)])
    plsc.subcore_barrier()
    sl = npad // NSUB
    lax.fori_loop(
        0, sl // L,
        lambda i, _: (accbuf.__setitem__(pl.ds(i * L, L), _zeros16()), 0)[1],
        0)
    for t in range(NSUB):
        pltpu.sync_copy(slab.at[pl.ds(t * IND_NP + s * sl, sl)],
                        tmpv.at[pl.ds(0, sl)])

        def addb(i, _):
            accbuf[pl.ds(i * L, L)] = (accbuf[pl.ds(i * L, L)]
                                       + tmpv[pl.ds(i * L, L)])
            return 0

        lax.fori_loop(0, sl // L, addb, 0)
    pltpu.sync_copy(accbuf.at[pl.ds(0, sl)],
                    out.at[pl.ds(c * npad + s * sl, sl)])
    plsc.subcore_barrier()


def _counts_call(d_ii, d_fi, d_if, d_ff):
    mesh = plsc.VectorSubcoreMesh(core_axis_name="c", subcore_axis_name="s")

    @functools.partial(
        pl.kernel,
        out_type=(jax.ShapeDtypeStruct((NCORE * IND_NP,), f32),
                  jax.ShapeDtypeStruct((NCORE * IND_NP,), f32),
                  jax.ShapeDtypeStruct((NCORE * FAC_NP,), f32),
                  jax.ShapeDtypeStruct((NCORE * FAC_NP,), f32)),
        mesh=mesh,
        scratch_types=[
            pltpu.VMEM((B,), i32),
            pltpu.VMEM((IND_NP,), f32),
            pltpu.VMEM((IND_NP // NSUB,), f32),
            pltpu.VMEM((IND_NP // NSUB,), f32),
            pltpu.VMEM_SHARED((NSUB * IND_NP,), f32),
        ],
        compiler_params=pltpu.CompilerParams(needs_layout_passes=False),
    )
    def k(d_ii_h, d_fi_h, d_if_h, d_ff_h, o_ii, o_fi, o_if, o_ff,
          dstv, cnt_local, tmpv, accbuf, slab):
        c = lax.axis_index("c")
        s = lax.axis_index("s")
        _emit_counts(c, s, d_ii_h, o_ii, IND_NP, E_II // 32 // B,
                     dstv, cnt_local, tmpv, accbuf, slab)
        _emit_counts(c, s, d_fi_h, o_fi, IND_NP, E_FI // 32 // B,
                     dstv, cnt_local, tmpv, accbuf, slab)
        _emit_counts(c, s, d_if_h, o_if, FAC_NP, E_IF // 32 // B,
                     dstv, cnt_local, tmpv, accbuf, slab)
        _emit_counts(c, s, d_ff_h, o_ff, FAC_NP, E_FF // 32 // B,
                     dstv, cnt_local, tmpv, accbuf, slab)

    return k(d_ii, d_fi, d_if, d_ff)


def _zero_fill_rows(rows):
    def zb(g, _):
        for k in range(H // L):
            rows[g, pl.ds(k * L, L)] = _zeros16()
        return 0
    lax.fori_loop(0, G, zb, 0)


def _zero_rows(zeros_v, acc, base, count):
    off = 0
    while off < count:
        blk = min(G, count - off)
        pltpu.sync_copy(zeros_v.at[pl.ds(0, blk)],
                        acc.at[pl.ds(base + off, blk)])
        off += blk


def _emit_job(c, s, hsrc, src_e, dst_e, out, nchunks, chunk, nblocks,
              srcv, dstv, srcv2, dstv2, cbs, cbd, gidx, gdst, rows, acc,
              sem, semla, semlb, semsc):
    tile_e = src_e.shape[0] // NSUB
    span = chunk // NSUB
    lane = lax.iota(i32, L)
    npairs = nblocks // 2

    def chunk_body(r, _):
        lo = (nchunks * c + r) * chunk
        hi = lo + chunk
        _zero_fill_rows(rows)
        for k in range(G // L):
            gdst[pl.ds(k * L, L)] = jnp.full((L,), GARBAGE, i32)
        pltpu.async_copy(rows, acc.at[gdst], semsc, add=True)
        _zero_rows(rows, acc, s * span, span)
        plsc.subcore_barrier()
        tbase = s * tile_e

        def wait_scatter():
            pltpu.make_async_copy(rows, acc.at[gdst], semsc).wait()

        def drain(carry):
            n, d = carry
            wait_scatter()
            for k in range(G // L):
                gidx[pl.ds(k * L, L)] = cbs[pl.ds(k * L, L)]
                gdst[pl.ds(k * L, L)] = cbd[pl.ds(k * L, L)]
            pltpu.async_copy(hsrc.at[gidx], rows, sem).wait()
            pltpu.async_copy(rows, acc.at[gdst], semsc, add=True)
            for k in range((CB - G) // L):
                cbs[pl.ds(k * L, L)] = cbs[pl.ds(G + k * L, L)]
                cbd[pl.ds(k * L, L)] = cbd[pl.ds(G + k * L, L)]
            return n - G, d + 1

        def compact(sbuf, dbuf, carry):
            n, d = carry
            for k in range(B // L):
                sv = sbuf[pl.ds(k * L, L)]
                dv = dbuf[pl.ds(k * L, L)]
                m = (dv >= lo) & (dv < hi)
                plsc.store_compressed(cbs.at[pl.ds(n, L)], sv, mask=m)
                plsc.store_compressed(cbd.at[pl.ds(n, L)], dv - lo, mask=m)
                n = n + jnp.sum(m.astype(i32))
            return lax.while_loop(lambda q: q[0] >= G, drain, (n, d))

        def issue_load(ebase, sbuf, dbuf, seml):
            pltpu.async_copy(src_e.at[pl.ds(ebase, B)], sbuf, seml)
            pltpu.async_copy(dst_e.at[pl.ds(ebase, B)], dbuf, seml)

        def wait_load(sbuf, dbuf, seml):
            pltpu.make_async_copy(src_e.at[pl.ds(0, B)], sbuf, seml).wait()
            pltpu.make_async_copy(dst_e.at[pl.ds(0, B)], dbuf, seml).wait()

        issue_load(tbase, srcv, dstv, semla)

        def pair_body(bi, carry):
            issue_load(tbase + (2 * bi + 1) * B, srcv2, dstv2, semlb)
            wait_load(srcv, dstv, semla)
            carry = compact(srcv, dstv, carry)

            @pl.when(bi + 1 < npairs)
            def _():
                issue_load(tbase + (2 * bi + 2) * B, srcv, dstv, semla)

            wait_load(srcv2, dstv2, semlb)
            return compact(srcv2, dstv2, carry)

        n, d = lax.fori_loop(0, npairs, pair_body,
                             (jnp.asarray(0, i32), jnp.asarray(0, i32)))
        wait_scatter()
        for k in range(G // L):
            m = (k * L + lane) < n
            gidx[pl.ds(k * L, L)] = jnp.where(m, cbs[pl.ds(k * L, L)],
                                              jnp.zeros((L,), i32))
            gdst[pl.ds(k * L, L)] = jnp.where(m, cbd[pl.ds(k * L, L)],
                                              jnp.full((L,), GARBAGE, i32))
        pltpu.async_copy(hsrc.at[gidx], rows, sem).wait()
        pltpu.sync_copy(rows, acc.at[gdst], add=True)
        plsc.subcore_barrier()
        pltpu.sync_copy(acc.at[pl.ds(s * span, span)],
                        out.at[pl.ds(lo + s * span, span)])
        plsc.subcore_barrier()
        return 0

    lax.fori_loop(0, nchunks, chunk_body, 0)


def _agg_call(h_i, h_f, s_ii, d_ii, s_fi, d_fi, s_if, d_if, s_ff, d_ff):
    mesh = plsc.VectorSubcoreMesh(core_axis_name="c", subcore_axis_name="s")

    @functools.partial(
        pl.kernel,
        out_type=(jax.ShapeDtypeStruct((IND_NP, H), f32),
                  jax.ShapeDtypeStruct((IND_NP, H), f32),
                  jax.ShapeDtypeStruct((FAC_NP, H), f32),
                  jax.ShapeDtypeStruct((FAC_NP, H), f32)),
        mesh=mesh,
        scratch_types=[
            pltpu.VMEM((B,), i32),
            pltpu.VMEM((B,), i32),
            pltpu.VMEM((B,), i32),
            pltpu.VMEM((B,), i32),
            pltpu.VMEM((CB,), i32),
            pltpu.VMEM((CB,), i32),
            pltpu.VMEM((G,), i32),
            pltpu.VMEM((G,), i32),
            pltpu.VMEM((G, H), f32),
            pltpu.VMEM_SHARED((ACC_ROWS, H), f32),
            pltpu.SemaphoreType.DMA,
            pltpu.SemaphoreType.DMA,
            pltpu.SemaphoreType.DMA,
            pltpu.SemaphoreType.DMA,
        ],
        compiler_params=pltpu.CompilerParams(needs_layout_passes=False),
    )
    def k(hi_h, hf_h, sii, dii, sfi, dfi, sif, dif, sff, dff,
          a_ii, a_fi, a_if, a_ff,
          srcv, dstv, srcv2, dstv2, cbs, cbd, gidx, gdst, rows, acc,
          sem, semla, semlb, semsc):
        c = lax.axis_index("c")
        s = lax.axis_index("s")
        scr = (srcv, dstv, srcv2, dstv2, cbs, cbd, gidx, gdst, rows, acc,
               sem, semla, semlb, semsc)
        _emit_job(c, s, hi_h, sii, dii, a_ii, 2, IND_CHUNK, E_II // NSUB // B,
                  *scr)
        _emit_job(c, s, hf_h, sfi, dfi, a_fi, 2, IND_CHUNK, E_FI // NSUB // B,
                  *scr)
        _emit_job(c, s, hi_h, sif, dif, a_if, 1, FAC_CHUNK, E_IF // NSUB // B,
                  *scr)
        _emit_job(c, s, hf_h, sff, dff, a_ff, 1, FAC_CHUNK, E_FF // NSUB // B,
                  *scr)

    return k(h_i, h_f, s_ii, d_ii, s_fi, d_fi, s_if, d_if, s_ff, d_ff)


def _proj_body(x_ref, w_ref, b_ref, out_ref):
    out_ref[...] = (jnp.dot(x_ref[...], w_ref[...],
                            preferred_element_type=f32) + b_ref[...])


def _proj_call(x, w, b):
    n = x.shape[0]
    r = 512
    return pl.pallas_call(
        _proj_body,
        grid=(n // r,),
        in_specs=[pl.BlockSpec((r, 8), lambda i: (i, 0)),
                  pl.BlockSpec((8, H), lambda i: (0, 0)),
                  pl.BlockSpec((1, H), lambda i: (0, 0))],
        out_specs=pl.BlockSpec((r, H), lambda i: (i, 0)),
        out_shape=jax.ShapeDtypeStruct((n, H), f32),
    )(x, w, b)


def _dense_body(h_ref, aa_ref, ab_ref, ia_ref, ib_ref, wla_ref, wlb_ref,
                wr_ref, bias_ref, g_ref, bln_ref, out_ref):
    h = h_ref[...]
    t = (jnp.dot(aa_ref[...] * ia_ref[...], wla_ref[...],
                 preferred_element_type=f32)
         + jnp.dot(ab_ref[...] * ib_ref[...], wlb_ref[...],
                   preferred_element_type=f32)
         + jnp.dot(h, wr_ref[...], preferred_element_type=f32)
         + bias_ref[...])
    mu = jnp.mean(t, axis=-1, keepdims=True)
    d = t - mu
    var = jnp.mean(d * d, axis=-1, keepdims=True)
    ln = g_ref[...] * d * lax.rsqrt(var + 1e-5) + bln_ref[...]
    out_ref[...] = jnp.maximum(h + ln, 0.0)


def _dense_call(h, aa, ab, ia, ib, wla, wlb, wr, bias, g, bln):
    n = h.shape[0]
    r = 512
    row = lambda i: (i, 0)
    fix = lambda i: (0, 0)
    return pl.pallas_call(
        _dense_body,
        grid=(n // r,),
        in_specs=[pl.BlockSpec((r, H), row), pl.BlockSpec((r, H), row),
                  pl.BlockSpec((r, H), row), pl.BlockSpec((r, 1), row),
                  pl.BlockSpec((r, 1), row), pl.BlockSpec((H, H), fix),
                  pl.BlockSpec((H, H), fix), pl.BlockSpec((H, H), fix),
                  pl.BlockSpec((1, H), fix), pl.BlockSpec((1, H), fix),
                  pl.BlockSpec((1, H), fix)],
        out_specs=pl.BlockSpec((r, H), row),
        out_shape=jax.ShapeDtypeStruct((n, H), f32),
    )(h, aa, ab, ia, ib, wla, wlb, wr, bias, g, bln)


def _pad_edges(src, dst, epad):
    e = src.shape[0]
    src = jnp.concatenate([src, jnp.zeros((epad - e,), i32)])
    dst = jnp.concatenate([dst, jnp.full((epad - e,), -1, i32)])
    return src, dst


def kernel(x_individual, x_facility, ei_interacts, ei_visits, ei_visited_by,
           ei_connects, params):
    p = params
    xi = jnp.zeros((IND_NP, 8), f32).at[:IND_N, :5].set(x_individual)
    xf = jnp.zeros((FAC_NP, 8), f32).at[:FAC_N, :3].set(x_facility)
    wi = jnp.zeros((8, H), f32).at[:5].set(p['ind_proj_W'])
    wf = jnp.zeros((8, H), f32).at[:3].set(p['fac_proj_W'])

    s_ii, d_ii = _pad_edges(ei_interacts[0], ei_interacts[1], E_II)
    s_fi, d_fi = _pad_edges(ei_visited_by[0], ei_visited_by[1], E_FI)
    s_if, d_if = _pad_edges(ei_visits[0], ei_visits[1], E_IF)
    s_ff, d_ff = _pad_edges(ei_connects[0], ei_connects[1], E_FF)

    c_ii, c_fi, c_if, c_ff = _counts_call(d_ii, d_fi, d_if, d_ff)
    inv = lambda c, npad: (1.0 / jnp.maximum(
        c.reshape(NCORE, npad).sum(0), 1.0))[:, None]
    inv_ii, inv_fi = inv(c_ii, IND_NP), inv(c_fi, IND_NP)
    inv_if, inv_ff = inv(c_if, FAC_NP), inv(c_ff, FAC_NP)

    h_i = _proj_call(xi, wi, p['ind_proj_b'][None])
    h_f = _proj_call(xf, wf, p['fac_proj_b'][None])

    for l in range(3):
        a_ii, a_fi, a_if, a_ff = _agg_call(
            h_i, h_f, s_ii, d_ii, s_fi, d_fi, s_if, d_if, s_ff, d_ff)
        h_i = _dense_call(
            h_i, a_ii, a_fi, inv_ii, inv_fi,
            p[f'ii{l}_Wl'], p[f'fi{l}_Wl'], p[f'ii{l}_Wr'] + p[f'fi{l}_Wr'],
            (p[f'ii{l}_bl'] + p[f'fi{l}_bl'])[None],
            p[f'ln_ind{l}_g'][None], p[f'ln_ind{l}_b'][None])
        h_f = _dense_call(
            h_f, a_if, a_ff, inv_if, inv_ff,
            p[f'if{l}_Wl'], p[f'ff{l}_Wl'], p[f'if{l}_Wr'] + p[f'ff{l}_Wr'],
            (p[f'if{l}_bl'] + p[f'ff{l}_bl'])[None],
            p[f'ln_fac{l}_g'][None], p[f'ln_fac{l}_b'][None])

    return h_i[:IND_N], h_f[:FAC_N]
